# Initial kernel scaffold; baseline (speedup 1.0000x reference)
import jax
import jax.numpy as jnp
from jax.experimental import pallas as pl


def kernel(x_nchw, fc_w, fc_b, p0_w, p0_gamma, p0_beta, p0_mean, p0_var, p1_w, p1_gamma, p1_beta, p1_mean, p1_var, p2_w, p2_gamma, p2_beta, p2_mean, p2_var, p3_w, p3_gamma, p3_beta, p3_mean, p3_var, p4_w, p4_gamma, p4_beta, p4_mean, p4_var, p5_w, p5_gamma, p5_beta, p5_mean, p5_var, p6_w, p6_gamma, p6_beta, p6_mean, p6_var, p7_w, p7_gamma, p7_beta, p7_mean, p7_var, p8_w, p8_gamma, p8_beta, p8_mean, p8_var, p9_w, p9_gamma, p9_beta, p9_mean, p9_var, p10_w, p10_gamma, p10_beta, p10_mean, p10_var):
    raise NotImplementedError("write your pallas kernel here")


# speedup vs baseline: 4.7564x; 4.7564x over previous
import functools

import jax
import jax.numpy as jnp
from jax import lax
from jax.experimental import pallas as pl
from jax.experimental.pallas import tpu as pltpu

_BN_EPS = 1e-5
_NB = 8


def _dw3x3(xp_ref, y, wdw_ref, dwsh_ref, *, H, W, C, stride):
    nb = y.shape[0]
    xp_ref[...] = jnp.zeros_like(xp_ref)
    xp_ref[:, 1:1 + H, 1:1 + W, :] = y
    Ho = (H - 1) // stride + 1
    Wo = (W - 1) // stride + 1
    acc = jnp.zeros((nb, Ho, Wo, C), jnp.float32)
    for t in range(9):
        di, dj = t // 3, t % 3
        if stride == 1:
            xs = xp_ref[:, di:di + H, dj:dj + W, :]
        else:
            xs = xp_ref[:, di:di + stride * (Ho - 1) + 1:stride,
                        dj:dj + stride * (Wo - 1) + 1:stride, :]
        acc = acc + xs * wdw_ref[t]
    return jnp.maximum(acc + dwsh_ref[...], 0.0)


def _pw1x1(y, pww_ref, pwsh_ref):
    nb, H, W, C = y.shape
    cout = pww_ref.shape[1]
    z = jnp.dot(y.reshape(nb * H * W, C), pww_ref[...],
                preferred_element_type=jnp.float32)
    z = jnp.maximum(z + pwsh_ref[...], 0.0)
    return z.reshape(nb, H, W, cout)


def _net_kernel(x_ref, w4_ref, sh0_ref,
                dw1_ref, ds1_ref, pw1_ref, ps1_ref,
                dw2_ref, ds2_ref, pw2_ref, ps2_ref,
                dw3_ref, ds3_ref, pw3_ref, ps3_ref,
                dw4_ref, ds4_ref, pw4_ref, ps4_ref,
                dw5_ref, ds5_ref, pw5_ref, ps5_ref,
                fcw_ref, fcb_ref, o_ref,
                xp1_ref, xp2_ref, xp3_ref, xp4_ref, xp5_ref):
    nb = x_ref.shape[0]

    acc = jnp.zeros((nb * 32 * 32, 32), jnp.float32)
    for k in range(4):
        ai, aj = k // 2, k % 2
        xs = x_ref[:, ai:ai + 32, aj:aj + 32, :].reshape(nb * 32 * 32, 12)
        acc = acc + jnp.dot(xs, w4_ref[k], preferred_element_type=jnp.float32)
    y = jnp.maximum(acc + sh0_ref[...], 0.0).reshape(nb, 32, 32, 32)

    y = _dw3x3(xp1_ref, y, dw1_ref, ds1_ref, H=32, W=32, C=32, stride=1)
    y = _pw1x1(y, pw1_ref, ps1_ref)
    y = _dw3x3(xp2_ref, y, dw2_ref, ds2_ref, H=32, W=32, C=64, stride=2)
    y = _pw1x1(y, pw2_ref, ps2_ref)
    y = _dw3x3(xp3_ref, y, dw3_ref, ds3_ref, H=16, W=16, C=128, stride=1)
    y = _pw1x1(y, pw3_ref, ps3_ref)
    y = _dw3x3(xp4_ref, y, dw4_ref, ds4_ref, H=16, W=16, C=128, stride=2)
    y = _pw1x1(y, pw4_ref, ps4_ref)
    y = _dw3x3(xp5_ref, y, dw5_ref, ds5_ref, H=8, W=8, C=256, stride=1)
    y = _pw1x1(y, pw5_ref, ps5_ref)

    pooled = jnp.sum(y.reshape(nb, 64, 256), axis=1)
    o_ref[...] = (jnp.dot(pooled, fcw_ref[...],
                          preferred_element_type=jnp.float32) + fcb_ref[...])


def _bn_fold(gamma, beta, mean, var):
    scale = gamma / jnp.sqrt(var + _BN_EPS)
    return scale, beta - mean * scale


def _prep_dws(dw_w, dw_g, dw_b, dw_m, dw_v, pw_w, pw_g, pw_b, pw_m, pw_v):
    sc, sh = _bn_fold(dw_g, dw_b, dw_m, dw_v)
    C = dw_w.shape[-1]
    wdw = dw_w.reshape(9, C) * sc[None, :]
    dwsh = sh.reshape(1, C)
    sc2, sh2 = _bn_fold(pw_g, pw_b, pw_m, pw_v)
    cout = pw_w.shape[-1]
    pww = pw_w.reshape(C, cout) * sc2[None, :]
    pwsh = sh2.reshape(1, cout)
    return wdw, dwsh, pww, pwsh


def kernel(x_nchw, fc_w, fc_b, p0_w, p0_gamma, p0_beta, p0_mean, p0_var, p1_w, p1_gamma, p1_beta, p1_mean, p1_var, p2_w, p2_gamma, p2_beta, p2_mean, p2_var, p3_w, p3_gamma, p3_beta, p3_mean, p3_var, p4_w, p4_gamma, p4_beta, p4_mean, p4_var, p5_w, p5_gamma, p5_beta, p5_mean, p5_var, p6_w, p6_gamma, p6_beta, p6_mean, p6_var, p7_w, p7_gamma, p7_beta, p7_mean, p7_var, p8_w, p8_gamma, p8_beta, p8_mean, p8_var, p9_w, p9_gamma, p9_beta, p9_mean, p9_var, p10_w, p10_gamma, p10_beta, p10_mean, p10_var):
    N = x_nchw.shape[0]
    nb = _NB

    x = jnp.transpose(x_nchw, (0, 2, 3, 1))
    x = x.reshape(N, 33, 2, 33, 2, 3).transpose(0, 1, 3, 2, 4, 5)
    x = x.reshape(N, 33, 33, 12)

    sc0, sh0 = _bn_fold(p0_gamma, p0_beta, p0_mean, p0_var)
    ws = p0_w * sc0
    w4 = jnp.zeros((2, 2, 12, 32), jnp.float32)
    for di in range(3):
        for dj in range(3):
            lo = (di % 2) * 6 + (dj % 2) * 3
            w4 = w4.at[di // 2, dj // 2, lo:lo + 3, :].set(ws[di, dj])
    w4 = w4.reshape(4, 12, 32)
    sh0 = sh0.reshape(1, 32)

    dw1, ds1, pw1, ps1 = _prep_dws(p1_w, p1_gamma, p1_beta, p1_mean, p1_var,
                                   p2_w, p2_gamma, p2_beta, p2_mean, p2_var)
    dw2, ds2, pw2, ps2 = _prep_dws(p3_w, p3_gamma, p3_beta, p3_mean, p3_var,
                                   p4_w, p4_gamma, p4_beta, p4_mean, p4_var)
    dw3, ds3, pw3, ps3 = _prep_dws(p5_w, p5_gamma, p5_beta, p5_mean, p5_var,
                                   p6_w, p6_gamma, p6_beta, p6_mean, p6_var)
    dw4, ds4, pw4, ps4 = _prep_dws(p7_w, p7_gamma, p7_beta, p7_mean, p7_var,
                                   p8_w, p8_gamma, p8_beta, p8_mean, p8_var)
    dw5, ds5, pw5, ps5 = _prep_dws(p9_w, p9_gamma, p9_beta, p9_mean, p9_var,
                                   p10_w, p10_gamma, p10_beta, p10_mean, p10_var)

    fcw = fc_w * (1.0 / 64.0)
    fcb = fc_b.reshape(1, -1)

    def _w(shape):
        nd = len(shape)
        return pl.BlockSpec(shape, lambda n, _nd=nd: (0,) * _nd)

    weight_args = [w4, sh0,
                   dw1, ds1, pw1, ps1,
                   dw2, ds2, pw2, ps2,
                   dw3, ds3, pw3, ps3,
                   dw4, ds4, pw4, ps4,
                   dw5, ds5, pw5, ps5,
                   fcw, fcb]

    out = pl.pallas_call(
        _net_kernel,
        out_shape=jax.ShapeDtypeStruct((N, 128), jnp.float32),
        grid=(N // nb,),
        in_specs=[pl.BlockSpec((nb, 33, 33, 12), lambda n: (n, 0, 0, 0))]
                 + [_w(a.shape) for a in weight_args],
        out_specs=pl.BlockSpec((nb, 128), lambda n: (n, 0)),
        scratch_shapes=[
            pltpu.VMEM((nb, 34, 34, 32), jnp.float32),
            pltpu.VMEM((nb, 34, 34, 64), jnp.float32),
            pltpu.VMEM((nb, 18, 18, 128), jnp.float32),
            pltpu.VMEM((nb, 18, 18, 128), jnp.float32),
            pltpu.VMEM((nb, 10, 10, 256), jnp.float32),
        ],
        compiler_params=pltpu.CompilerParams(
            dimension_semantics=("parallel",)),
    )(x, *weight_args)
    return out
